# Initial kernel scaffold; baseline (speedup 1.0000x reference)
import jax
import jax.numpy as jnp
from jax.experimental import pallas as pl


def kernel(sample, entity_embedding, relation_embedding):
    raise NotImplementedError("write your pallas kernel here")


# speedup vs baseline: 1.3654x; 1.3654x over previous
import functools

import jax
import jax.numpy as jnp
from jax import lax
from jax.experimental import pallas as pl
from jax.experimental.pallas import tpu as pltpu
from jax.experimental.pallas import tpu_sc as plsc

GAMMA = 12.0
B = 16384
DIM = 128
NC = 2
NS = 16
L = 16
NW = NC * NS
BPW = B // NW
C = 128
NCH = BPW // C
G = DIM // L


def _sc_score(heads, rels, tails, entity, relation):
    mesh = plsc.VectorSubcoreMesh(core_axis_name="c", subcore_axis_name="s")

    @functools.partial(
        pl.kernel,
        mesh=mesh,
        compiler_params=pltpu.CompilerParams(needs_layout_passes=False),
        out_type=jax.ShapeDtypeStruct((B,), jnp.float32),
        scratch_types=[
            pltpu.VMEM((NCH, C), jnp.int32),
            pltpu.VMEM((NCH, C), jnp.int32),
            pltpu.VMEM((NCH, C), jnp.int32),
            pltpu.VMEM((C, DIM), jnp.float32),
            pltpu.VMEM((C, DIM), jnp.float32),
            pltpu.VMEM((C, DIM), jnp.float32),
            pltpu.VMEM((L * L,), jnp.float32),
            pltpu.VMEM((BPW,), jnp.float32),
            pltpu.SemaphoreType.DMA,
        ],
    )
    def body(heads_hbm, rels_hbm, tails_hbm, ent_hbm, rel_hbm, out_hbm,
             h_idx, r_idx, t_idx, h_buf, r_buf, t_buf, tr, scores, sem):
        wid = lax.axis_index("s") * NC + lax.axis_index("c")
        base = wid * BPW

        for c in range(NCH):
            pltpu.sync_copy(heads_hbm.at[pl.ds(base + c * C, C)], h_idx.at[c])
            pltpu.sync_copy(rels_hbm.at[pl.ds(base + c * C, C)], r_idx.at[c])
            pltpu.sync_copy(tails_hbm.at[pl.ds(base + c * C, C)], t_idx.at[c])

        def chunk_body(c, carry):
            cp_h = pltpu.async_copy(ent_hbm.at[h_idx.at[c]], h_buf, sem)
            cp_r = pltpu.async_copy(rel_hbm.at[r_idx.at[c]], r_buf, sem)
            cp_t = pltpu.async_copy(ent_hbm.at[t_idx.at[c]], t_buf, sem)
            cp_h.wait()
            cp_r.wait()
            cp_t.wait()

            def grp_body(g2, carry2):
                r0 = g2 * L
                lane = lax.iota(jnp.int32, L)
                for j in range(L):
                    acc = jnp.zeros((L,), jnp.float32)
                    for g in range(G):
                        h = h_buf[r0 + j, pl.ds(g * L, L)]
                        r = r_buf[r0 + j, pl.ds(g * L, L)]
                        t = t_buf[r0 + j, pl.ds(g * L, L)]
                        acc = acc + jnp.abs(h + r - t)
                    tr[pl.ds(j * L, L)] = acc
                sv = jnp.zeros((L,), jnp.float32)
                for i in range(L):
                    sv = sv + plsc.load_gather(tr, [lane * L + i])
                scores[pl.ds(c * C + r0, L)] = GAMMA - sv
                return carry2

            lax.fori_loop(0, C // L, grp_body, 0)
            return carry

        lax.fori_loop(0, NCH, chunk_body, 0)
        pltpu.sync_copy(scores, out_hbm.at[pl.ds(base, BPW)])

    return body(heads, rels, tails, entity, relation)


def kernel(sample, entity_embedding, relation_embedding):
    heads = sample[:, 0]
    rels = sample[:, 1]
    tails = sample[:, 2]
    scores = _sc_score(heads, rels, tails, entity_embedding, relation_embedding)
    return scores.reshape(B, 1)
